# baseline (speedup 1.0000x reference)
import functools

import jax
import jax.numpy as jnp
from jax import lax
from jax.experimental import pallas as pl
from jax.experimental.pallas import tpu as pltpu
from jax.experimental.pallas import tpu_sc as plsc

_B, _N, _CODE_DIM, _K, _HIDDEN = 16, 1024, 256, 8192, 768
_COMMITMENT_COST = 0.25
_THRESHOLD = 0.5

_TOK = _B * _N


_info = plsc.get_sparse_core_info()
_NC, _NS = _info.num_cores, _info.num_subcores
_NW = _NC * _NS
_PER_W = _TOK // _NW
_CH = 128
_N_CH = _PER_W // _CH
_NB = 3


def _make_sc_gather():
    mesh = plsc.VectorSubcoreMesh(core_axis_name="c", subcore_axis_name="s")

    @functools.partial(
        pl.kernel,
        mesh=mesh,
        out_type=jax.ShapeDtypeStruct((_TOK, _CODE_DIM), jnp.float32),
        scratch_types=[
            pltpu.VMEM((_N_CH, _CH), jnp.int32),
            pltpu.VMEM((_CH, _CODE_DIM), jnp.float32),
            pltpu.VMEM((_CH, _CODE_DIM), jnp.float32),
            pltpu.VMEM((_CH, _CODE_DIM), jnp.float32),
            pltpu.SemaphoreType.DMA,
            pltpu.SemaphoreType.DMA,
            pltpu.SemaphoreType.DMA,
            pltpu.SemaphoreType.DMA,
            pltpu.SemaphoreType.DMA,
            pltpu.SemaphoreType.DMA,
        ],
    )
    def sc_gather(table_hbm, idx_hbm, out_hbm, idx_v,
                  rows0, rows1, rows2, g0, g1, g2, w0, w1, w2):
        wid = lax.axis_index("s") * _NC + lax.axis_index("c")
        base = wid * _PER_W
        pltpu.sync_copy(idx_hbm.at[wid], idx_v)
        bufs = (rows0, rows1, rows2)
        gsems = (g0, g1, g2)
        wsems = (w0, w1, w2)
        gcopies = [None] * _NB
        wcopies = [None] * _N_CH
        for c in range(_NB):
            gcopies[c] = pltpu.async_copy(
                table_hbm.at[idx_v.at[c]], bufs[c], gsems[c])
        for c in range(_N_CH):
            bi = c % _NB
            gcopies[bi].wait()
            wcopies[c] = pltpu.async_copy(
                bufs[bi], out_hbm.at[pl.ds(base + c * _CH, _CH)], wsems[bi])
            nc = c + _NB
            if nc < _N_CH:
                wcopies[nc - _NB].wait()
                wcopies[nc - _NB] = None
                gcopies[bi] = pltpu.async_copy(
                    table_hbm.at[idx_v.at[nc]], bufs[bi], gsems[bi])
        for c in range(_N_CH):
            if wcopies[c] is not None:
                wcopies[c].wait()

    return sc_gather


_sc_gather = _make_sc_gather()


_SUB = _N // 128
_BB = 2
_STEPS = _B // _BB


def _allmax(x):
    for s in (1, 2, 4):
        x = jnp.maximum(x, pltpu.roll(x, s, 0))
    for s in (1, 2, 4, 8, 16, 32, 64):
        x = jnp.maximum(x, pltpu.roll(x, s, 1))
    return x


def _allsum(x):
    for s in (1, 2, 4):
        x = x + pltpu.roll(x, s, 0)
    for s in (1, 2, 4, 8, 16, 32, 64):
        x = x + pltpu.roll(x, s, 1)
    return x


def _tc_body(q_ref, m_ref, w_ref, b_ref, o_ref, valid_ref, loss_ref, acc_ref):
    bidx = pl.program_id(0)
    q2 = q_ref[...].reshape(_BB * _N, _CODE_DIM)
    o_ref[...] = (lax.dot_general(
        q2, w_ref[...], (((1,), (1,)), ((), ())),
        preferred_element_type=jnp.float32) + b_ref[...]).reshape(
            _BB, _N, _HIDDEN)

    q4 = q_ref[...].reshape(_BB, _SUB, 128, _CODE_DIM)
    m4 = m_ref[...].reshape(_BB, _SUB, 128, _CODE_DIM)
    num_v = jnp.zeros((_SUB, 128), jnp.float32)
    cnt_v = jnp.zeros((_SUB, 128), jnp.float32)
    for r in range(_BB):
        sim = jnp.sum(q4[r] * m4[r], axis=2)
        sq = jnp.sum((m4[r] - q4[r]) ** 2, axis=2)

        neg = jnp.float32(-jnp.inf)
        cur = jnp.full((_SUB, 128), jnp.inf, jnp.float32)
        removed = jnp.zeros((_SUB, 128), jnp.float32)
        for _ in range(5):
            mmax = _allmax(jnp.where(sim < cur, sim, neg))
            cnt_eq = _allsum(jnp.where(sim == mmax, 1.0, 0.0))
            upd = removed < 5.0
            removed = jnp.where(upd, removed + cnt_eq, removed)
            cur = jnp.where(upd, mmax, cur)

        thresh = jnp.minimum(cur, jnp.float32(_THRESHOLD))
        validf = (sim >= thresh).astype(jnp.float32)
        valid_ref[r] = validf.astype(jnp.int32)

        num_v = num_v + sq * validf
        cnt_v = cnt_v + validf

    @pl.when(bidx == 0)
    def _init():
        acc_ref[0] = num_v
        acc_ref[1] = cnt_v

    @pl.when(bidx > 0)
    def _accum():
        acc_ref[0] = acc_ref[0] + num_v
        acc_ref[1] = acc_ref[1] + cnt_v

    @pl.when(bidx == _STEPS - 1)
    def _final():
        num = jnp.sum(acc_ref[0])
        denom = jnp.sum(acc_ref[1]) * jnp.float32(_CODE_DIM)
        loss = (1.0 + _COMMITMENT_COST) * num / denom
        loss_ref[...] = jnp.full((1, 1), loss, jnp.float32)


_tc_call = pl.pallas_call(
    _tc_body,
    grid=(_STEPS,),
    in_specs=[
        pl.BlockSpec((_BB, _N, _CODE_DIM), lambda b: (b, 0, 0)),
        pl.BlockSpec((_BB, _N, _CODE_DIM), lambda b: (b, 0, 0)),
        pl.BlockSpec((_HIDDEN, _CODE_DIM), lambda b: (0, 0)),
        pl.BlockSpec((1, _HIDDEN), lambda b: (0, 0)),
    ],
    out_specs=[
        pl.BlockSpec((_BB, _N, _HIDDEN), lambda b: (b, 0, 0)),
        pl.BlockSpec((_BB, _SUB, 128), lambda b: (b, 0, 0)),
        pl.BlockSpec((1, 1), lambda b: (0, 0)),
    ],
    out_shape=[
        jax.ShapeDtypeStruct((_B, _N, _HIDDEN), jnp.float32),
        jax.ShapeDtypeStruct((_B, _SUB, 128), jnp.int32),
        jax.ShapeDtypeStruct((1, 1), jnp.float32),
    ],
    scratch_shapes=[pltpu.VMEM((2, _SUB, 128), jnp.float32)],
)


def kernel(mlc_proj, code, code_id, W, b):
    idx = code_id.reshape(_NW, _N_CH, _CH).astype(jnp.int32)
    quant_flat = _sc_gather(code, idx)
    quant = quant_flat.reshape(_B, _N, _CODE_DIM)
    out, valid3, loss = _tc_call(quant, mlc_proj, W, b.reshape(1, _HIDDEN))
    valid = valid3.reshape(_B, _N) != 0
    return out, valid, loss.reshape(())


# speedup vs baseline: 9.7806x; 9.7806x over previous
import functools

import jax
import jax.numpy as jnp
from jax import lax
from jax.experimental import pallas as pl
from jax.experimental.pallas import tpu as pltpu
from jax.experimental.pallas import tpu_sc as plsc

_B, _N, _CODE_DIM, _K, _HIDDEN = 16, 1024, 256, 8192, 768
_COMMITMENT_COST = 0.25
_THRESHOLD = 0.5

_TOK = _B * _N


_info = plsc.get_sparse_core_info()
_NC, _NS = _info.num_cores, _info.num_subcores
_NW = _NC * _NS
_PER_W = _TOK // _NW
_CH = 128
_N_CH = _PER_W // _CH
_NB = 3


def _make_sc_gather():
    mesh = plsc.VectorSubcoreMesh(core_axis_name="c", subcore_axis_name="s")

    @functools.partial(
        pl.kernel,
        mesh=mesh,
        out_type=jax.ShapeDtypeStruct((_TOK, _CODE_DIM), jnp.float32),
        scratch_types=[
            pltpu.VMEM((_N_CH, _CH), jnp.int32),
            pltpu.VMEM((_CH, _CODE_DIM), jnp.float32),
            pltpu.VMEM((_CH, _CODE_DIM), jnp.float32),
            pltpu.VMEM((_CH, _CODE_DIM), jnp.float32),
            pltpu.SemaphoreType.DMA,
            pltpu.SemaphoreType.DMA,
            pltpu.SemaphoreType.DMA,
            pltpu.SemaphoreType.DMA,
            pltpu.SemaphoreType.DMA,
            pltpu.SemaphoreType.DMA,
        ],
    )
    def sc_gather(table_hbm, idx_hbm, out_hbm, idx_v,
                  rows0, rows1, rows2, g0, g1, g2, w0, w1, w2):
        wid = lax.axis_index("s") * _NC + lax.axis_index("c")
        base = wid * _PER_W
        pltpu.sync_copy(idx_hbm.at[wid], idx_v)
        bufs = (rows0, rows1, rows2)
        gsems = (g0, g1, g2)
        wsems = (w0, w1, w2)
        gcopies = [None] * _NB
        wcopies = [None] * _N_CH
        for c in range(_NB):
            gcopies[c] = pltpu.async_copy(
                table_hbm.at[idx_v.at[c]], bufs[c], gsems[c])
        for c in range(_N_CH):
            bi = c % _NB
            gcopies[bi].wait()
            wcopies[c] = pltpu.async_copy(
                bufs[bi], out_hbm.at[pl.ds(base + c * _CH, _CH)], wsems[bi])
            nc = c + _NB
            if nc < _N_CH:
                wcopies[nc - _NB].wait()
                wcopies[nc - _NB] = None
                gcopies[bi] = pltpu.async_copy(
                    table_hbm.at[idx_v.at[nc]], bufs[bi], gsems[bi])
        for c in range(_N_CH):
            if wcopies[c] is not None:
                wcopies[c].wait()

    return sc_gather


_sc_gather = _make_sc_gather()


_SUB = _N // 128
_BB = 2
_STEPS = _B // _BB


def _tc_body(q_ref, m_ref, w_ref, b_ref, o_ref, valid_ref, loss_ref, acc_ref):
    bidx = pl.program_id(0)
    q2 = q_ref[...].reshape(_BB * _N, _CODE_DIM)
    o_ref[...] = (lax.dot_general(
        q2, w_ref[...], (((1,), (1,)), ((), ())),
        preferred_element_type=jnp.float32) + b_ref[...]).reshape(
            _BB, _N, _HIDDEN)

    q4 = q_ref[...].reshape(_BB, _SUB, 128, _CODE_DIM)
    m4 = m_ref[...].reshape(_BB, _SUB, 128, _CODE_DIM)
    num_v = jnp.zeros((_SUB, 128), jnp.float32)
    cnt_v = jnp.zeros((_SUB, 128), jnp.float32)
    for r in range(_BB):
        sim = jnp.sum(q4[r] * m4[r], axis=2)
        sq = jnp.sum((m4[r] - q4[r]) ** 2, axis=2)

        neg = jnp.float32(-jnp.inf)
        cur = jnp.float32(jnp.inf)
        vs = []
        for _ in range(5):
            cur = jnp.max(jnp.where(sim < cur, sim, neg))
            vs.append(cur)
        cnts = [jnp.sum((sim >= v).astype(jnp.float32)) for v in vs[:4]]
        thresh = vs[4]
        for i in (3, 2, 1, 0):
            thresh = jnp.where(cnts[i] >= 5.0, vs[i], thresh)
        thresh = jnp.minimum(thresh, jnp.float32(_THRESHOLD))
        validf = (sim >= thresh).astype(jnp.float32)
        valid_ref[r] = validf.astype(jnp.int32)

        num_v = num_v + sq * validf
        cnt_v = cnt_v + validf

    @pl.when(bidx == 0)
    def _init():
        acc_ref[0] = num_v
        acc_ref[1] = cnt_v

    @pl.when(bidx > 0)
    def _accum():
        acc_ref[0] = acc_ref[0] + num_v
        acc_ref[1] = acc_ref[1] + cnt_v

    @pl.when(bidx == _STEPS - 1)
    def _final():
        num = jnp.sum(acc_ref[0])
        denom = jnp.sum(acc_ref[1]) * jnp.float32(_CODE_DIM)
        loss = (1.0 + _COMMITMENT_COST) * num / denom
        loss_ref[...] = jnp.full((1, 1), loss, jnp.float32)


_tc_call = pl.pallas_call(
    _tc_body,
    grid=(_STEPS,),
    in_specs=[
        pl.BlockSpec((_BB, _N, _CODE_DIM), lambda b: (b, 0, 0)),
        pl.BlockSpec((_BB, _N, _CODE_DIM), lambda b: (b, 0, 0)),
        pl.BlockSpec((_HIDDEN, _CODE_DIM), lambda b: (0, 0)),
        pl.BlockSpec((1, _HIDDEN), lambda b: (0, 0)),
    ],
    out_specs=[
        pl.BlockSpec((_BB, _N, _HIDDEN), lambda b: (b, 0, 0)),
        pl.BlockSpec((_BB, _SUB, 128), lambda b: (b, 0, 0)),
        pl.BlockSpec((1, 1), lambda b: (0, 0)),
    ],
    out_shape=[
        jax.ShapeDtypeStruct((_B, _N, _HIDDEN), jnp.float32),
        jax.ShapeDtypeStruct((_B, _SUB, 128), jnp.int32),
        jax.ShapeDtypeStruct((1, 1), jnp.float32),
    ],
    scratch_shapes=[pltpu.VMEM((2, _SUB, 128), jnp.float32)],
)


def kernel(mlc_proj, code, code_id, W, b):
    idx = code_id.reshape(_NW, _N_CH, _CH).astype(jnp.int32)
    quant_flat = _sc_gather(code, idx)
    quant = quant_flat.reshape(_B, _N, _CODE_DIM)
    out, valid3, loss = _tc_call(quant, mlc_proj, W, b.reshape(1, _HIDDEN))
    valid = valid3.reshape(_B, _N) != 0
    return out, valid, loss.reshape(())


# speedup vs baseline: 9.8958x; 1.0118x over previous
import functools

import jax
import jax.numpy as jnp
from jax import lax
from jax.experimental import pallas as pl
from jax.experimental.pallas import tpu as pltpu
from jax.experimental.pallas import tpu_sc as plsc

_B, _N, _CODE_DIM, _K, _HIDDEN = 16, 1024, 256, 8192, 768
_COMMITMENT_COST = 0.25
_THRESHOLD = 0.5

_TOK = _B * _N


_info = plsc.get_sparse_core_info()
_NC, _NS = _info.num_cores, _info.num_subcores
_NW = _NC * _NS
_PER_W = _TOK // _NW
_CH = 128
_N_CH = _PER_W // _CH
_NB = 3


def _make_sc_gather():
    mesh = plsc.VectorSubcoreMesh(core_axis_name="c", subcore_axis_name="s")

    @functools.partial(
        pl.kernel,
        mesh=mesh,
        out_type=jax.ShapeDtypeStruct((_TOK, _CODE_DIM), jnp.float32),
        scratch_types=[
            pltpu.VMEM((_N_CH, _CH), jnp.int32),
            pltpu.VMEM((_CH, _CODE_DIM), jnp.float32),
            pltpu.VMEM((_CH, _CODE_DIM), jnp.float32),
            pltpu.VMEM((_CH, _CODE_DIM), jnp.float32),
            pltpu.SemaphoreType.DMA,
            pltpu.SemaphoreType.DMA,
            pltpu.SemaphoreType.DMA,
            pltpu.SemaphoreType.DMA,
            pltpu.SemaphoreType.DMA,
            pltpu.SemaphoreType.DMA,
        ],
    )
    def sc_gather(table_hbm, idx_hbm, out_hbm, idx_v,
                  rows0, rows1, rows2, g0, g1, g2, w0, w1, w2):
        wid = lax.axis_index("s") * _NC + lax.axis_index("c")
        base = wid * _PER_W
        pltpu.sync_copy(idx_hbm.at[wid], idx_v)
        bufs = (rows0, rows1, rows2)
        gsems = (g0, g1, g2)
        wsems = (w0, w1, w2)
        gcopies = [None] * _NB
        wcopies = [None] * _N_CH
        for c in range(_NB):
            gcopies[c] = pltpu.async_copy(
                table_hbm.at[idx_v.at[c]], bufs[c], gsems[c])
        for c in range(_N_CH):
            bi = c % _NB
            gcopies[bi].wait()
            wcopies[c] = pltpu.async_copy(
                bufs[bi], out_hbm.at[pl.ds(base + c * _CH, _CH)], wsems[bi])
            nc = c + _NB
            if nc < _N_CH:
                wcopies[nc - _NB].wait()
                wcopies[nc - _NB] = None
                gcopies[bi] = pltpu.async_copy(
                    table_hbm.at[idx_v.at[nc]], bufs[bi], gsems[bi])
        for c in range(_N_CH):
            if wcopies[c] is not None:
                wcopies[c].wait()

    return sc_gather


_sc_gather = _make_sc_gather()


_SUB = _N // 128
_BB = 4
_STEPS = _B // _BB


def _tc_body(q_ref, m_ref, w_ref, b_ref, o_ref, valid_ref, loss_ref, acc_ref):
    bidx = pl.program_id(0)
    q2 = q_ref[...].reshape(_BB * _N, _CODE_DIM)
    o_ref[...] = (lax.dot_general(
        q2, w_ref[...], (((1,), (1,)), ((), ())),
        preferred_element_type=jnp.float32) + b_ref[...]).reshape(
            _BB, _N, _HIDDEN)

    q4 = q_ref[...].reshape(_BB, _SUB, 128, _CODE_DIM)
    m4 = m_ref[...].reshape(_BB, _SUB, 128, _CODE_DIM)
    num_v = jnp.zeros((_SUB, 128), jnp.float32)
    cnt_v = jnp.zeros((_SUB, 128), jnp.float32)
    for r in range(_BB):
        sim = jnp.sum(q4[r] * m4[r], axis=2)
        sq = jnp.sum((m4[r] - q4[r]) ** 2, axis=2)

        neg = jnp.float32(-jnp.inf)
        cur = jnp.float32(jnp.inf)
        vs = []
        for _ in range(5):
            cur = jnp.max(jnp.where(sim < cur, sim, neg))
            vs.append(cur)
        cnts = [jnp.sum((sim >= v).astype(jnp.float32)) for v in vs[:4]]
        thresh = vs[4]
        for i in (3, 2, 1, 0):
            thresh = jnp.where(cnts[i] >= 5.0, vs[i], thresh)
        thresh = jnp.minimum(thresh, jnp.float32(_THRESHOLD))
        validf = (sim >= thresh).astype(jnp.float32)
        valid_ref[r] = validf.astype(jnp.int32)

        num_v = num_v + sq * validf
        cnt_v = cnt_v + validf

    @pl.when(bidx == 0)
    def _init():
        acc_ref[0] = num_v
        acc_ref[1] = cnt_v

    @pl.when(bidx > 0)
    def _accum():
        acc_ref[0] = acc_ref[0] + num_v
        acc_ref[1] = acc_ref[1] + cnt_v

    @pl.when(bidx == _STEPS - 1)
    def _final():
        num = jnp.sum(acc_ref[0])
        denom = jnp.sum(acc_ref[1]) * jnp.float32(_CODE_DIM)
        loss = (1.0 + _COMMITMENT_COST) * num / denom
        loss_ref[...] = jnp.full((1, 1), loss, jnp.float32)


_tc_call = pl.pallas_call(
    _tc_body,
    grid=(_STEPS,),
    in_specs=[
        pl.BlockSpec((_BB, _N, _CODE_DIM), lambda b: (b, 0, 0)),
        pl.BlockSpec((_BB, _N, _CODE_DIM), lambda b: (b, 0, 0)),
        pl.BlockSpec((_HIDDEN, _CODE_DIM), lambda b: (0, 0)),
        pl.BlockSpec((1, _HIDDEN), lambda b: (0, 0)),
    ],
    out_specs=[
        pl.BlockSpec((_BB, _N, _HIDDEN), lambda b: (b, 0, 0)),
        pl.BlockSpec((_BB, _SUB, 128), lambda b: (b, 0, 0)),
        pl.BlockSpec((1, 1), lambda b: (0, 0)),
    ],
    out_shape=[
        jax.ShapeDtypeStruct((_B, _N, _HIDDEN), jnp.float32),
        jax.ShapeDtypeStruct((_B, _SUB, 128), jnp.int32),
        jax.ShapeDtypeStruct((1, 1), jnp.float32),
    ],
    scratch_shapes=[pltpu.VMEM((2, _SUB, 128), jnp.float32)],
)


def kernel(mlc_proj, code, code_id, W, b):
    idx = code_id.reshape(_NW, _N_CH, _CH).astype(jnp.int32)
    quant_flat = _sc_gather(code, idx)
    quant = quant_flat.reshape(_B, _N, _CODE_DIM)
    out, valid3, loss = _tc_call(quant, mlc_proj, W, b.reshape(1, _HIDDEN))
    valid = valid3.reshape(_B, _N) != 0
    return out, valid, loss.reshape(())
